# Initial kernel scaffold; baseline (speedup 1.0000x reference)
import jax
import jax.numpy as jnp
from jax.experimental import pallas as pl


def kernel(hidden_states, Wq, bq, Wk, bk, Wv, bv, Wout, bout):
    raise NotImplementedError("write your pallas kernel here")


# speedup vs baseline: 1.5421x; 1.5421x over previous
import functools

import jax
import jax.numpy as jnp
from jax import lax
from jax.experimental import pallas as pl
from jax.experimental.pallas import tpu as pltpu
from jax.experimental.pallas import tpu_sc as plsc

F32 = jnp.float32
BF16 = jnp.bfloat16

H = 16
H2O_RATIO = 0.2
QB = 256
IB = 256
RB = 256


def _proj_body(hsb_ref, wb_ref, b_ref, out_ref, *, sfact):
    acc = jnp.dot(hsb_ref[...], wb_ref[...], preferred_element_type=F32)
    out_ref[...] = (acc + b_ref[...]) * F32(sfact)


def _attn_body(q_ref, k_ref, v_ref, o_ref, aggr_ref):
    q = q_ref[...].astype(BF16)
    k = k_ref[0, 0].astype(BF16)
    v = v_ref[0, 0].astype(BF16)
    s = lax.dot_general(q, k, (((1,), (1,)), ((), ())),
                        preferred_element_type=F32)
    rows = lax.broadcasted_iota(jnp.int32, s.shape, 0)
    cols = lax.broadcasted_iota(jnp.int32, s.shape, 1)
    s = s + jnp.where(cols > rows, F32(-10000.0), F32(0.0))
    m = jnp.max(s, axis=1, keepdims=True)
    e = jnp.exp(s - m)
    p = e / jnp.sum(e, axis=1, keepdims=True)
    o_ref[...] = jnp.dot(p.astype(BF16), v, preferred_element_type=F32)
    aggr_ref[0] = jnp.sum(p, axis=0, keepdims=True)


def _rank_body(acol_ref, arow_ref, rank_ref):
    ib = pl.program_id(1)
    ai = acol_ref[0]
    aj = arow_ref[0]
    shape = (ai.shape[0], aj.shape[1])
    i_idx = lax.broadcasted_iota(jnp.int32, shape, 0) + ib * IB
    j_idx = lax.broadcasted_iota(jnp.int32, shape, 1)
    beats = (ai > aj) | ((ai == aj) & (i_idx < j_idx))
    part = jnp.sum(beats.astype(jnp.int32), axis=0, keepdims=True)

    @pl.when(ib == 0)
    def _():
        rank_ref[0] = part

    @pl.when(ib > 0)
    def _():
        rank_ref[0] = rank_ref[0] + part


def _oproj_body(o_ref, wt_ref, b_ref, out_ref):
    out_ref[...] = jnp.dot(o_ref[...].astype(BF16), wt_ref[...].astype(BF16),
                           preferred_element_type=F32) + b_ref[...]


def _sc_select_gather(rank1d, aggr1d, kv2d, T, hh):
    hd = kv2d.shape[1]
    hh_pad = (hh + 15) // 16 * 16
    mesh = plsc.VectorSubcoreMesh(core_axis_name="c", subcore_axis_name="s")
    chunks = []
    off = 0
    while off < hh_pad:
        ln = min(128, hh_pad - off)
        chunks.append((off, ln))
        off += ln

    @functools.partial(
        pl.kernel, mesh=mesh,
        compiler_params=pltpu.CompilerParams(needs_layout_passes=False),
        out_type=[
            jax.ShapeDtypeStruct((2, H, hh_pad, hd), F32),
            jax.ShapeDtypeStruct((H, hh_pad), F32),
        ],
        scratch_types=[
            pltpu.VMEM((T,), jnp.int32),
            pltpu.VMEM((T,), F32),
            pltpu.VMEM((hh_pad,), jnp.int32),
            pltpu.VMEM((hh_pad,), F32),
            pltpu.VMEM((128,), jnp.int32),
            pltpu.VMEM((hh_pad, hd), F32),
            pltpu.SemaphoreType.DMA,
        ],
    )
    def sc_kernel(rank_hbm, aggr_hbm, kv_hbm, rows_out, acc_out,
                  rank_v, aggr_v, tidx_v, tval_v, cidx_v, rows_v, sem):
        wid = lax.axis_index("s") * 2 + lax.axis_index("c")
        h = wid // 2
        j = wid % 2
        base = j * (H * T) + h * T
        pltpu.sync_copy(rank_hbm.at[pl.ds(h * T, T)], rank_v)
        pltpu.sync_copy(aggr_hbm.at[pl.ds(h * T, T)], aggr_v)

        zero16i = jnp.zeros((16,), jnp.int32)
        for z in range(hh_pad // 16):
            tidx_v[pl.ds(z * 16, 16)] = zero16i

        hh_i = jnp.int32(hh)

        def scat(i, carry):
            r = rank_v[pl.ds(i * 16, 16)]
            val = aggr_v[pl.ds(i * 16, 16)]
            gidx = lax.iota(jnp.int32, 16) + (i * 16 + base)
            rc = jnp.where(r < hh_i, r, hh_i)
            plsc.store_scatter(tidx_v, [rc], gidx)
            plsc.store_scatter(tval_v, [rc], val)
            return carry

        lax.fori_loop(0, T // 16, scat, 0)

        for off, ln in chunks:
            for z in range(ln // 16):
                cidx_v[pl.ds(z * 16, 16)] = tidx_v[pl.ds(off + z * 16, 16)]
            pltpu.async_copy(
                kv_hbm.at[cidx_v.at[pl.ds(0, ln)]],
                rows_v.at[pl.ds(off, ln)], sem).wait()
        pltpu.sync_copy(rows_v, rows_out.at[j, h])

        @pl.when(j == 0)
        def _():
            pltpu.sync_copy(tval_v, acc_out.at[h])

    return sc_kernel(rank1d, aggr1d, kv2d)


def kernel(hidden_states, Wq, bq, Wk, bk, Wv, bv, Wout, bout):
    Bsz, T, D = hidden_states.shape
    hd = D // H
    hh = int(T * H2O_RATIO)
    hh_pad = (hh + 15) // 16 * 16
    hs = hidden_states.reshape(T, D)
    nqb, nib, nrb = T // QB, T // IB, T // RB

    hs_bf = hs.astype(BF16)

    def _proj(wt_bf, b, sfact):
        return pl.pallas_call(
            functools.partial(_proj_body, sfact=sfact),
            in_specs=[
                pl.BlockSpec((T, D), lambda: (0, 0)),
                pl.BlockSpec((D, D), lambda: (0, 0)),
                pl.BlockSpec((1, D), lambda: (0, 0)),
            ],
            out_specs=pl.BlockSpec((T, D), lambda: (0, 0)),
            out_shape=jax.ShapeDtypeStruct((T, D), F32),
        )(hs_bf, wt_bf, b)

    q2d = _proj(Wq.T.astype(BF16), bq.reshape(1, D), hd ** (-0.5))
    k2d = _proj(Wk.T.astype(BF16), bk.reshape(1, D), 1.0)
    v2d = _proj(Wv.T.astype(BF16), bv.reshape(1, D), 1.0)
    kv_heads = jnp.stack([k2d, v2d]).reshape(2, T, H, hd).transpose(0, 2, 1, 3)

    o_merged, aggr = pl.pallas_call(
        _attn_body,
        grid=(H,),
        in_specs=[
            pl.BlockSpec((T, hd), lambda h: (0, h)),
            pl.BlockSpec((1, 1, T, hd), lambda h: (0, h, 0, 0)),
            pl.BlockSpec((1, 1, T, hd), lambda h: (1, h, 0, 0)),
        ],
        out_specs=[
            pl.BlockSpec((T, hd), lambda h: (0, h)),
            pl.BlockSpec((1, 1, T), lambda h: (h, 0, 0)),
        ],
        out_shape=[
            jax.ShapeDtypeStruct((T, D), F32),
            jax.ShapeDtypeStruct((H, 1, T), F32),
        ],
    )(q2d, kv_heads, kv_heads)

    rank = pl.pallas_call(
        _rank_body,
        grid=(H, nib),
        in_specs=[
            pl.BlockSpec((1, IB, 1), lambda h, ib: (h, ib, 0)),
            pl.BlockSpec((1, 1, T), lambda h, ib: (h, 0, 0)),
        ],
        out_specs=pl.BlockSpec((1, 1, T), lambda h, ib: (h, 0, 0)),
        out_shape=jax.ShapeDtypeStruct((H, 1, T), jnp.int32),
    )(aggr.reshape(H, T, 1), aggr)

    attn_out = pl.pallas_call(
        _oproj_body,
        grid=(nrb,),
        in_specs=[
            pl.BlockSpec((RB, D), lambda r: (r, 0)),
            pl.BlockSpec((D, D), lambda r: (0, 0)),
            pl.BlockSpec((1, D), lambda r: (0, 0)),
        ],
        out_specs=pl.BlockSpec((RB, D), lambda r: (r, 0)),
        out_shape=jax.ShapeDtypeStruct((T, D), F32),
    )(o_merged, Wout.T, bout.reshape(1, D))

    rows, acc_pad = _sc_select_gather(
        rank.reshape(H * T), aggr.reshape(H * T),
        kv_heads.reshape(2 * H * T, hd), T, hh)

    past_k = jnp.concatenate(
        [rows[0, :, :hh][None], jnp.zeros((1, H, 1, hd), F32)], axis=2)
    past_v = jnp.concatenate(
        [rows[1, :, :hh][None], jnp.zeros((1, H, 1, hd), F32)], axis=2)
    acc = acc_pad[:, :hh].T
    return attn_out.reshape(Bsz, T, D), past_k, past_v, acc


# speedup vs baseline: 1.8026x; 1.1689x over previous
import functools

import jax
import jax.numpy as jnp
from jax import lax
from jax.experimental import pallas as pl
from jax.experimental.pallas import tpu as pltpu
from jax.experimental.pallas import tpu_sc as plsc

F32 = jnp.float32
BF16 = jnp.bfloat16

H = 16
H2O_RATIO = 0.2
QB = 256
IB = 256
RB = 256


def _proj_body(hsb_ref, wb_ref, b_ref, out_ref, *, sfact):
    acc = jnp.dot(hsb_ref[...], wb_ref[...], preferred_element_type=F32)
    out_ref[...] = (acc + b_ref[...]) * F32(sfact)


def _proj_heads_body(hsb_ref, wb_ref, b_ref, out_ref):
    hd = out_ref.shape[-1]
    acc = jnp.dot(hsb_ref[...], wb_ref[...], preferred_element_type=F32)
    val = acc + b_ref[...]
    for h in range(H):
        out_ref[h] = val[:, h * hd:(h + 1) * hd]


def _attn_body(q_ref, k_ref, v_ref, o_ref, aggr_ref):
    q = q_ref[...].astype(BF16)
    k = k_ref[0].astype(BF16)
    v = v_ref[0].astype(BF16)
    s = lax.dot_general(q, k, (((1,), (1,)), ((), ())),
                        preferred_element_type=F32)
    rows = lax.broadcasted_iota(jnp.int32, s.shape, 0)
    cols = lax.broadcasted_iota(jnp.int32, s.shape, 1)
    s = s + jnp.where(cols > rows, F32(-10000.0), F32(0.0))
    m = jnp.max(s, axis=1, keepdims=True)
    e = jnp.exp(s - m)
    p = e / jnp.sum(e, axis=1, keepdims=True)
    o_ref[...] = jnp.dot(p.astype(BF16), v, preferred_element_type=F32)
    aggr_ref[0] = jnp.sum(p, axis=0, keepdims=True)


def _rank_body(acol_ref, arow_ref, rank_ref):
    ib = pl.program_id(1)
    ai = acol_ref[0]
    aj = arow_ref[0]
    shape = (ai.shape[0], aj.shape[1])
    i_idx = lax.broadcasted_iota(jnp.int32, shape, 0) + ib * IB
    j_idx = lax.broadcasted_iota(jnp.int32, shape, 1)
    beats = (ai > aj) | ((ai == aj) & (i_idx < j_idx))
    part = jnp.sum(beats.astype(jnp.int32), axis=0, keepdims=True)

    @pl.when(ib == 0)
    def _():
        rank_ref[0] = part

    @pl.when(ib > 0)
    def _():
        rank_ref[0] = rank_ref[0] + part


def _oproj_body(o_ref, wt_ref, b_ref, out_ref):
    out_ref[...] = jnp.dot(o_ref[...].astype(BF16), wt_ref[...].astype(BF16),
                           preferred_element_type=F32) + b_ref[...]


def _sc_select_gather(rank1d, aggr1d, k2d, v2d, T, hh):
    hd = k2d.shape[1]
    hh_pad = (hh + 15) // 16 * 16
    mesh = plsc.VectorSubcoreMesh(core_axis_name="c", subcore_axis_name="s")
    chunks = []
    off = 0
    while off < hh_pad:
        ln = min(128, hh_pad - off)
        chunks.append((off, ln))
        off += ln

    @functools.partial(
        pl.kernel, mesh=mesh,
        compiler_params=pltpu.CompilerParams(needs_layout_passes=False),
        out_type=[
            jax.ShapeDtypeStruct((2, H, hh_pad, hd), F32),
            jax.ShapeDtypeStruct((H, hh_pad), F32),
        ],
        scratch_types=[
            pltpu.VMEM((T,), jnp.int32),
            pltpu.VMEM((T,), F32),
            pltpu.VMEM((hh_pad,), jnp.int32),
            pltpu.VMEM((hh_pad,), F32),
            pltpu.VMEM((128,), jnp.int32),
            pltpu.VMEM((hh_pad, hd), F32),
            pltpu.SemaphoreType.DMA,
        ],
    )
    def sc_kernel(rank_hbm, aggr_hbm, k_hbm, v_hbm, rows_out, acc_out,
                  rank_v, aggr_v, tidx_v, tval_v, cidx_v, rows_v, sem):
        wid = lax.axis_index("s") * 2 + lax.axis_index("c")
        h = wid // 2
        j = wid % 2
        base = h * T
        pltpu.sync_copy(rank_hbm.at[pl.ds(h * T, T)], rank_v)
        pltpu.sync_copy(aggr_hbm.at[pl.ds(h * T, T)], aggr_v)

        zero16i = jnp.zeros((16,), jnp.int32)
        for z in range(hh_pad // 16):
            tidx_v[pl.ds(z * 16, 16)] = zero16i

        hh_i = jnp.int32(hh)

        def scat(i, carry):
            r = rank_v[pl.ds(i * 16, 16)]
            val = aggr_v[pl.ds(i * 16, 16)]
            gidx = lax.iota(jnp.int32, 16) + (i * 16 + base)
            rc = jnp.where(r < hh_i, r, hh_i)
            plsc.store_scatter(tidx_v, [rc], gidx)
            plsc.store_scatter(tval_v, [rc], val)
            return carry

        lax.fori_loop(0, T // 16, scat, 0)

        for off, ln in chunks:
            for z in range(ln // 16):
                cidx_v[pl.ds(z * 16, 16)] = tidx_v[pl.ds(off + z * 16, 16)]
            cslice = cidx_v.at[pl.ds(0, ln)]
            rslice = rows_v.at[pl.ds(off, ln)]

            @pl.when(j == 0)
            def _():
                pltpu.async_copy(k_hbm.at[cslice], rslice, sem).wait()

            @pl.when(j == 1)
            def _():
                pltpu.async_copy(v_hbm.at[cslice], rslice, sem).wait()

        pltpu.sync_copy(rows_v, rows_out.at[j, h])

        @pl.when(j == 0)
        def _():
            pltpu.sync_copy(tval_v, acc_out.at[h])

    return sc_kernel(rank1d, aggr1d, k2d, v2d)


def kernel(hidden_states, Wq, bq, Wk, bk, Wv, bv, Wout, bout):
    Bsz, T, D = hidden_states.shape
    hd = D // H
    hh = int(T * H2O_RATIO)
    hh_pad = (hh + 15) // 16 * 16
    hs = hidden_states.reshape(T, D)
    nqb, nib, nrb = T // QB, T // IB, T // RB

    hs_bf = hs.astype(BF16)

    def _proj(wt_bf, b, sfact):
        return pl.pallas_call(
            functools.partial(_proj_body, sfact=sfact),
            in_specs=[
                pl.BlockSpec((T, D), lambda: (0, 0)),
                pl.BlockSpec((D, D), lambda: (0, 0)),
                pl.BlockSpec((1, D), lambda: (0, 0)),
            ],
            out_specs=pl.BlockSpec((T, D), lambda: (0, 0)),
            out_shape=jax.ShapeDtypeStruct((T, D), F32),
        )(hs_bf, wt_bf, b)

    q2d = _proj(Wq.T.astype(BF16), bq.reshape(1, D), hd ** (-0.5))

    def _proj_heads(wt_bf, b):
        return pl.pallas_call(
            _proj_heads_body,
            in_specs=[
                pl.BlockSpec((T, D), lambda: (0, 0)),
                pl.BlockSpec((D, D), lambda: (0, 0)),
                pl.BlockSpec((1, D), lambda: (0, 0)),
            ],
            out_specs=pl.BlockSpec((H, T, hd), lambda: (0, 0, 0)),
            out_shape=jax.ShapeDtypeStruct((H, T, hd), F32),
        )(hs_bf, wt_bf, b)

    k_heads = _proj_heads(Wk.T.astype(BF16), bk.reshape(1, D))
    v_heads = _proj_heads(Wv.T.astype(BF16), bv.reshape(1, D))

    o_merged, aggr = pl.pallas_call(
        _attn_body,
        grid=(H,),
        in_specs=[
            pl.BlockSpec((T, hd), lambda h: (0, h)),
            pl.BlockSpec((1, T, hd), lambda h: (h, 0, 0)),
            pl.BlockSpec((1, T, hd), lambda h: (h, 0, 0)),
        ],
        out_specs=[
            pl.BlockSpec((T, hd), lambda h: (0, h)),
            pl.BlockSpec((1, 1, T), lambda h: (h, 0, 0)),
        ],
        out_shape=[
            jax.ShapeDtypeStruct((T, D), F32),
            jax.ShapeDtypeStruct((H, 1, T), F32),
        ],
    )(q2d, k_heads, v_heads)

    rank = pl.pallas_call(
        _rank_body,
        grid=(H, nib),
        in_specs=[
            pl.BlockSpec((1, IB, 1), lambda h, ib: (h, ib, 0)),
            pl.BlockSpec((1, 1, T), lambda h, ib: (h, 0, 0)),
        ],
        out_specs=pl.BlockSpec((1, 1, T), lambda h, ib: (h, 0, 0)),
        out_shape=jax.ShapeDtypeStruct((H, 1, T), jnp.int32),
    )(aggr.reshape(H, T, 1), aggr)

    attn_out = pl.pallas_call(
        _oproj_body,
        grid=(nrb,),
        in_specs=[
            pl.BlockSpec((RB, D), lambda r: (r, 0)),
            pl.BlockSpec((D, D), lambda r: (0, 0)),
            pl.BlockSpec((1, D), lambda r: (0, 0)),
        ],
        out_specs=pl.BlockSpec((RB, D), lambda r: (r, 0)),
        out_shape=jax.ShapeDtypeStruct((T, D), F32),
    )(o_merged, Wout.T, bout.reshape(1, D))

    rows, acc_pad = _sc_select_gather(
        rank.reshape(H * T), aggr.reshape(H * T),
        k_heads.reshape(H * T, hd), v_heads.reshape(H * T, hd), T, hh)

    past_k = jnp.concatenate(
        [rows[0, :, :hh][None], jnp.zeros((1, H, 1, hd), F32)], axis=2)
    past_v = jnp.concatenate(
        [rows[1, :, :hh][None], jnp.zeros((1, H, 1, hd), F32)], axis=2)
    acc = acc_pad[:, :hh].T
    return attn_out.reshape(Bsz, T, D), past_k, past_v, acc
